# Initial kernel scaffold; baseline (speedup 1.0000x reference)
import jax
import jax.numpy as jnp
from jax.experimental import pallas as pl


def kernel(union_box, box_features, agg_xyz, seed_feature, seed_xyz, box_feature_union, W, b):
    raise NotImplementedError("write your pallas kernel here")


# speedup vs baseline: 1.8124x; 1.8124x over previous
import functools

import jax
import jax.numpy as jnp
from jax import lax
from jax.experimental import pallas as pl

U = 256
N = 1024
P = 256
C = 256
D = 128
OUTD = 128
TU = 8


def _body(ub_ref, sxyz_ref, axyz_ref, sfT_ref, bf_ref, bfu_ref, w_ref, b_ref,
          out_ref):
    i = pl.program_id(0)
    u0 = i * TU
    ub = ub_ref[pl.ds(u0, TU), :]
    bmin = ub[:, 0:3] - 0.5 * ub[:, 3:6]
    bmax = ub[:, 0:3] + 0.5 * ub[:, 3:6]

    def inside(xyz_cols, npts):
        m = None
        for k in range(3):
            x = xyz_cols[k:k + 1, :]
            lo = bmin[:, k:k + 1]
            hi = bmax[:, k:k + 1]
            mk = (x >= lo) & (x <= hi)
            m = mk if m is None else (m & mk)
        return m

    mask_s = inside(sxyz_ref[:], N)
    mask_a = inside(axyz_ref[:], P)

    mf_s = mask_s.astype(jnp.float32)[:, :, None]
    g1 = jnp.max(mf_s * sfT_ref[:][None, :, :], axis=1)
    mf_a = mask_a.astype(jnp.float32)[:, :, None]
    g2 = jnp.max(mf_a * bf_ref[:][None, :, :], axis=1)

    glob = jnp.concatenate([g1, g2, bfu_ref[pl.ds(u0, TU), :]], axis=1)
    out = lax.dot_general(glob, w_ref[:], (((1,), (1,)), ((), ())),
                          preferred_element_type=jnp.float32)
    out = out + b_ref[:]
    out_ref[:] = jax.nn.sigmoid(jnp.log(jnp.abs(out + 1e-6)))


def kernel(union_box, box_features, agg_xyz, seed_feature, seed_xyz,
           box_feature_union, W, b):
    ub = union_box[0]
    sxyz = seed_xyz.T
    axyz = agg_xyz.T
    sfT = seed_feature.T
    bfu = box_feature_union[:, 0, :]
    b2 = b.reshape(1, OUTD)

    full = lambda shape: pl.BlockSpec(shape, lambda i: (0, 0))
    return pl.pallas_call(
        _body,
        grid=(U // TU,),
        in_specs=[
            full((U, 6)),
            full((3, N)),
            full((3, P)),
            full((N, C)),
            full((P, D)),
            full((U, D)),
            full((OUTD, C + D + D)),
            full((1, OUTD)),
        ],
        out_specs=pl.BlockSpec((TU, OUTD), lambda i: (i, 0)),
        out_shape=jax.ShapeDtypeStruct((U, OUTD), jnp.float32),
    )(ub, sxyz, axyz, sfT, box_features, bfu, W, b2)
